# Initial kernel scaffold; baseline (speedup 1.0000x reference)
import jax
import jax.numpy as jnp
from jax.experimental import pallas as pl


def kernel(x_query, W, prompt_memory, prompt_keys):
    raise NotImplementedError("write your pallas kernel here")


# speedup vs baseline: 1.6233x; 1.6233x over previous
import functools

import jax
import jax.numpy as jnp
from jax import lax
from jax.experimental import pallas as pl
from jax.experimental.pallas import tpu as pltpu
from jax.experimental.pallas import tpu_sc as plsc

B = 1024
E = 1024
KD = 256
M = 8192
PL = 8
K = 8
D = PL * E

NW = 32
RPW = B // NW


def _topk_body(x_ref, w_ref, keys_ref, idx_ref, wts_ref):
    x = x_ref[...]
    Wm = w_ref[...]
    proj = lax.dot_general(x, Wm, (((1,), (1,)), ((), ())),
                           preferred_element_type=jnp.float32)
    pn = jnp.sqrt(jnp.sum(proj * proj, axis=1, keepdims=True))
    proj = proj / jnp.maximum(pn, 1e-12)
    keys = keys_ref[...]
    kn = jnp.sqrt(jnp.sum(keys * keys, axis=1, keepdims=True))
    keys = keys / jnp.maximum(kn, 1e-12)
    sim = lax.dot_general(proj, keys, (((1,), (1,)), ((), ())),
                          preferred_element_type=jnp.float32)

    col = lax.broadcasted_iota(jnp.int32, sim.shape, 1)
    scores, idxs = [], []
    for _ in range(K):
        m = jnp.max(sim, axis=1, keepdims=True)
        am = jnp.min(jnp.where(sim == m, col, jnp.int32(M)), axis=1,
                     keepdims=True)
        scores.append(m)
        idxs.append(am)
        sim = jnp.where(col == am, jnp.float32(-jnp.inf), sim)
    s = jnp.concatenate(scores, axis=1)
    ii = jnp.concatenate(idxs, axis=1)
    w = jnp.exp(s - s[:, 0:1])
    w = w / jnp.sum(w, axis=1, keepdims=True)
    idx_ref[...] = ii
    wts_ref[...] = jnp.pad(w, ((0, 0), (0, 8)))


def _topk_call(x_query, W, prompt_keys):
    BB = 256
    grid = (B // BB,)
    return pl.pallas_call(
        _topk_body,
        grid=grid,
        in_specs=[
            pl.BlockSpec((BB, E), lambda i: (i, 0)),
            pl.BlockSpec((KD, E), lambda i: (0, 0)),
            pl.BlockSpec((M, KD), lambda i: (0, 0)),
        ],
        out_specs=[
            pl.BlockSpec((BB, K), lambda i: (i, 0)),
            pl.BlockSpec((BB, 16), lambda i: (i, 0)),
        ],
        out_shape=[
            jax.ShapeDtypeStruct((B, K), jnp.int32),
            jax.ShapeDtypeStruct((B, 16), jnp.float32),
        ],
    )(x_query, W, prompt_keys)


def _combine_body(pm_hbm, idx_hbm, wts_hbm, out_hbm,
                  idx_v, wts_v, rows_v, orow_v, sem):
    wid = lax.axis_index("s") * 2 + lax.axis_index("c")
    rowbase = wid * RPW
    pltpu.sync_copy(idx_hbm.at[pl.ds(rowbase, RPW)], idx_v)
    pltpu.sync_copy(wts_hbm.at[pl.ds(rowbase, RPW)], wts_v)

    def row_body(r, carry):
        pltpu.async_copy(pm_hbm.at[idx_v.at[r]], rows_v, sem).wait()
        wv = wts_v[r]
        wb = [wv[jnp.full((16,), k, jnp.int32)] for k in range(K)]

        def chunk(j, c):
            acc = rows_v[0, pl.ds(j * 16, 16)] * wb[0]
            for k in range(1, K):
                acc = acc + rows_v[k, pl.ds(j * 16, 16)] * wb[k]
            orow_v[pl.ds(j * 16, 16)] = acc
            return c

        lax.fori_loop(0, D // 16, chunk, 0)
        pltpu.sync_copy(orow_v, out_hbm.at[rowbase + r])
        return carry

    lax.fori_loop(0, RPW, row_body, 0)


def _combine_call(pm2, idx, wts):
    mesh = plsc.VectorSubcoreMesh(core_axis_name="c", subcore_axis_name="s")
    f = pl.kernel(
        _combine_body,
        out_type=jax.ShapeDtypeStruct((B, D), jnp.float32),
        mesh=mesh,
        scratch_types=[
            pltpu.VMEM((RPW, K), jnp.int32),
            pltpu.VMEM((RPW, 16), jnp.float32),
            pltpu.VMEM((K, D), jnp.float32),
            pltpu.VMEM((D,), jnp.float32),
            pltpu.SemaphoreType.DMA,
        ],
    )
    return f(pm2, idx, wts)


def kernel(x_query, W, prompt_memory, prompt_keys):
    idx, wts = _topk_call(x_query, W, prompt_keys)
    pm2 = prompt_memory.reshape(M, D)
    out = _combine_call(pm2, idx, wts)
    return out.reshape(B, PL, E)


# speedup vs baseline: 2.9551x; 1.8204x over previous
import functools

import jax
import jax.numpy as jnp
from jax import lax
from jax.experimental import pallas as pl
from jax.experimental.pallas import tpu as pltpu
from jax.experimental.pallas import tpu_sc as plsc

B = 1024
E = 1024
KD = 256
M = 8192
PL = 8
K = 8
D = PL * E

NW = 32
RPW = B // NW


def _topk_body(x_ref, w_ref, keys_ref, idx_ref, wts_ref):
    x = x_ref[...]
    Wm = w_ref[...]
    proj = lax.dot_general(x, Wm, (((1,), (1,)), ((), ())),
                           preferred_element_type=jnp.float32)
    pn = jnp.sqrt(jnp.sum(proj * proj, axis=1, keepdims=True))
    proj = proj / jnp.maximum(pn, 1e-12)
    keys = keys_ref[...]
    kn = jnp.sqrt(jnp.sum(keys * keys, axis=1, keepdims=True))
    keys = keys / jnp.maximum(kn, 1e-12)
    sim = lax.dot_general(proj, keys, (((1,), (1,)), ((), ())),
                          preferred_element_type=jnp.float32)

    col = lax.broadcasted_iota(jnp.int32, sim.shape, 1)
    scores, idxs = [], []
    for _ in range(K):
        m = jnp.max(sim, axis=1, keepdims=True)
        am = jnp.min(jnp.where(sim == m, col, jnp.int32(M)), axis=1,
                     keepdims=True)
        scores.append(m)
        idxs.append(am)
        sim = jnp.where(col == am, jnp.float32(-jnp.inf), sim)
    s = jnp.concatenate(scores, axis=1)
    ii = jnp.concatenate(idxs, axis=1)
    w = jnp.exp(s - s[:, 0:1])
    w = w / jnp.sum(w, axis=1, keepdims=True)
    idx_ref[...] = ii
    wts_ref[...] = jnp.pad(w, ((0, 0), (0, 8)))


def _topk_call(x_query, W, prompt_keys):
    BB = 256
    grid = (B // BB,)
    return pl.pallas_call(
        _topk_body,
        grid=grid,
        in_specs=[
            pl.BlockSpec((BB, E), lambda i: (i, 0)),
            pl.BlockSpec((KD, E), lambda i: (0, 0)),
            pl.BlockSpec((M, KD), lambda i: (0, 0)),
        ],
        out_specs=[
            pl.BlockSpec((BB, K), lambda i: (i, 0)),
            pl.BlockSpec((BB, 16), lambda i: (i, 0)),
        ],
        out_shape=[
            jax.ShapeDtypeStruct((B, K), jnp.int32),
            jax.ShapeDtypeStruct((B, 16), jnp.float32),
        ],
    )(x_query, W, prompt_keys)


def _combine_body(pm_hbm, idx_hbm, wts_hbm, out_hbm,
                  idx_v, wts_v, rows_v, orow_v, sem):
    wid = lax.axis_index("s") * 2 + lax.axis_index("c")
    rowbase = wid * RPW
    pltpu.sync_copy(idx_hbm.at[pl.ds(rowbase, RPW)], idx_v)
    pltpu.sync_copy(wts_hbm.at[pl.ds(rowbase, RPW)], wts_v)

    def row_body(r, carry):
        pltpu.async_copy(pm_hbm.at[idx_v.at[r]], rows_v, sem).wait()
        wv = wts_v[r]
        wb = [wv[jnp.full((16,), k, jnp.int32)] for k in range(K)]

        for p in range(PL):
            def chunk(j, c):
                acc = rows_v[0, p, pl.ds(j * 16, 16)] * wb[0]
                for k in range(1, K):
                    acc = acc + rows_v[k, p, pl.ds(j * 16, 16)] * wb[k]
                orow_v[p, pl.ds(j * 16, 16)] = acc
                return c

            lax.fori_loop(0, E // 16, chunk, 0)
        pltpu.sync_copy(orow_v, out_hbm.at[rowbase + r])
        return carry

    lax.fori_loop(0, RPW, row_body, 0)


def _combine_call(pm, idx, wts):
    mesh = plsc.VectorSubcoreMesh(core_axis_name="c", subcore_axis_name="s")
    f = pl.kernel(
        _combine_body,
        out_type=jax.ShapeDtypeStruct((B, PL, E), jnp.float32),
        mesh=mesh,
        scratch_types=[
            pltpu.VMEM((RPW, K), jnp.int32),
            pltpu.VMEM((RPW, 16), jnp.float32),
            pltpu.VMEM((K, PL, E), jnp.float32),
            pltpu.VMEM((PL, E), jnp.float32),
            pltpu.SemaphoreType.DMA,
        ],
    )
    return f(pm, idx, wts)


def kernel(x_query, W, prompt_memory, prompt_keys):
    idx, wts = _topk_call(x_query, W, prompt_keys)
    return _combine_call(prompt_memory, idx, wts)
